# initial kernel scaffold (unmeasured)
import functools

import jax
import jax.numpy as jnp
from jax import lax
from jax.experimental import pallas as pl
from jax.experimental.pallas import tpu as pltpu

N_DEV = 4
N_TOK = 2048
D = 1024
E_LOCAL = 8
N_EXP = 32
CHUNK = N_TOK // N_DEV


def kernel(x, router_W, route_idx, expert_W, shared_W):
    x_bf = x.astype(jnp.bfloat16)
    router_bf = router_W.astype(jnp.bfloat16)
    expert_bf = expert_W.astype(jnp.bfloat16)
    shared_bf = shared_W.astype(jnp.bfloat16)

    def body(
        x_ref,
        router_ref,
        route_ref,
        expert_ref,
        shared_ref,
        out_ref,
        stage,
        land,
        ag_own,
        ag_land,
        rs_send,
        rs_recv,
        ag_send,
        ag_recv,
    ):
        my = lax.axis_index("i")
        left = (my - 1) % N_DEV
        right = (my + 1) % N_DEV

        barrier = pltpu.get_barrier_semaphore()
        for nbr in (left, right):
            pl.semaphore_signal(
                barrier, inc=1, device_id=(nbr,),
                device_id_type=pl.DeviceIdType.MESH,
            )
        pl.semaphore_wait(barrier, 2)

        scores = jnp.dot(
            x_ref[...], router_ref[...], preferred_element_type=jnp.float32
        )
        s_max = jnp.max(scores, axis=-1, keepdims=True)
        e_un = jnp.exp(scores - s_max)
        probs = e_un / jnp.sum(e_un, axis=-1, keepdims=True)
        col = lax.broadcasted_iota(jnp.int32, (N_TOK, N_EXP), 1)
        gates = jnp.where(col == route_ref[...], probs, 0.0)

        acc = jnp.zeros((N_TOK, D), jnp.float32)
        for le in range(E_LOCAL):
            e_glob = my * E_LOCAL + le
            g = jnp.sum(
                jnp.where(col == e_glob, gates, 0.0), axis=-1, keepdims=True
            )
            part = jnp.dot(
                x_ref[...], expert_ref[le], preferred_element_type=jnp.float32
            )
            acc = acc + g * part
        out_ref[...] = acc

        for s in range(N_DEV - 1):
            c_send = (my - s) % N_DEV
            if s == 0:
                stage[0] = out_ref[pl.ds(c_send * CHUNK, CHUNK), :].astype(
                    jnp.bfloat16
                )
            rdma = pltpu.make_async_remote_copy(
                src_ref=stage.at[s],
                dst_ref=land.at[s],
                send_sem=rs_send.at[s],
                recv_sem=rs_recv.at[s],
                device_id=(right,),
                device_id_type=pl.DeviceIdType.MESH,
            )
            rdma.start()
            rdma.wait()
            c_recv = (my - 1 - s) % N_DEV
            merged = out_ref[pl.ds(c_recv * CHUNK, CHUNK), :] + land[s].astype(
                jnp.float32
            )
            out_ref[pl.ds(c_recv * CHUNK, CHUNK), :] = merged
            if s < N_DEV - 2:
                stage[s + 1] = merged.astype(jnp.bfloat16)

        q = (my + 1) % N_DEV
        shared_q = jnp.dot(
            x_ref[pl.ds(q * CHUNK, CHUNK), :],
            shared_ref[...],
            preferred_element_type=jnp.float32,
        )
        final_q = out_ref[pl.ds(q * CHUNK, CHUNK), :] + shared_q
        out_ref[pl.ds(q * CHUNK, CHUNK), :] = final_q
        ag_own[...] = final_q.astype(jnp.bfloat16)

        for s in range(N_DEV - 1):
            rdma = pltpu.make_async_remote_copy(
                src_ref=ag_own if s == 0 else ag_land.at[s - 1],
                dst_ref=ag_land.at[s],
                send_sem=ag_send.at[s],
                recv_sem=ag_recv.at[s],
                device_id=(right,),
                device_id_type=pl.DeviceIdType.MESH,
            )
            rdma.start()
            rdma.wait()
            c_recv = (my - s) % N_DEV
            out_ref[pl.ds(c_recv * CHUNK, CHUNK), :] = ag_land[s].astype(
                jnp.float32
            )

        @functools.partial(
            pl.run_scoped, second=pltpu.SemaphoreType.REGULAR
        )
        def _(second):
            for nbr in (left, right):
                pl.semaphore_signal(
                    second, inc=1, device_id=(nbr,),
                    device_id_type=pl.DeviceIdType.MESH,
                )
            pl.semaphore_wait(second, 2)

    return pl.pallas_call(
        body,
        out_shape=jax.ShapeDtypeStruct((N_TOK, D), jnp.float32),
        in_specs=[pl.BlockSpec(memory_space=pltpu.VMEM)] * 5,
        out_specs=pl.BlockSpec(memory_space=pltpu.VMEM),
        scratch_shapes=[
            pltpu.VMEM((N_DEV - 1, CHUNK, D), jnp.bfloat16),
            pltpu.VMEM((N_DEV - 1, CHUNK, D), jnp.bfloat16),
            pltpu.VMEM((CHUNK, D), jnp.bfloat16),
            pltpu.VMEM((N_DEV - 1, CHUNK, D), jnp.bfloat16),
            pltpu.SemaphoreType.DMA((N_DEV - 1,)),
            pltpu.SemaphoreType.DMA((N_DEV - 1,)),
            pltpu.SemaphoreType.DMA((N_DEV - 1,)),
            pltpu.SemaphoreType.DMA((N_DEV - 1,)),
        ],
        compiler_params=pltpu.CompilerParams(collective_id=0),
    )(x_bf, router_bf, route_idx, expert_bf, shared_bf)


# baseline (device time: 171077 ns/iter reference)
import functools

import jax
import jax.numpy as jnp
from jax import lax
from jax.experimental import pallas as pl
from jax.experimental.pallas import tpu as pltpu

N_DEV = 4
N_TOK = 2048
D = 1024
E_LOCAL = 8
N_EXP = 32
CHUNK = N_TOK // N_DEV


def kernel(x, router_W, route_idx, expert_W, shared_W):
    x_bf = x.astype(jnp.bfloat16)
    router_bf = router_W.astype(jnp.bfloat16)
    expert_bf = expert_W.astype(jnp.bfloat16)
    shared_bf = shared_W.astype(jnp.bfloat16)

    def body(
        x_ref,
        router_ref,
        route_ref,
        expert_ref,
        shared_ref,
        out_ref,
        stage,
        land,
        ag_own,
        ag_land,
        rs_send,
        rs_recv,
        ag_send,
        ag_recv,
    ):
        my = lax.axis_index("i")
        left = (my - 1) % N_DEV
        right = (my + 1) % N_DEV

        barrier = pltpu.get_barrier_semaphore()
        for nbr in (left, right):
            pl.semaphore_signal(
                barrier, inc=1, device_id=(nbr,),
                device_id_type=pl.DeviceIdType.MESH,
            )
        pl.semaphore_wait(barrier, 2)

        scores = jnp.dot(
            x_ref[...], router_ref[...], preferred_element_type=jnp.float32
        )
        s_max = jnp.max(scores, axis=-1, keepdims=True)
        e_un = jnp.exp(scores - s_max)
        probs = e_un / jnp.sum(e_un, axis=-1, keepdims=True)
        col = lax.broadcasted_iota(jnp.int32, (N_TOK, N_EXP), 1)
        gates = jnp.where(col == route_ref[...], probs, 0.0)

        for cc in range(N_DEV):
            r0, r1 = cc * CHUNK, (cc + 1) * CHUNK
            xc = x_ref[r0:r1, :]
            acc_c = jnp.zeros((CHUNK, D), jnp.float32)
            for le in range(E_LOCAL):
                e_glob = my * E_LOCAL + le
                g = jnp.sum(
                    jnp.where(col[r0:r1, :] == e_glob, gates[r0:r1, :], 0.0),
                    axis=-1,
                    keepdims=True,
                )
                part = jnp.dot(
                    xc, expert_ref[le], preferred_element_type=jnp.float32
                )
                acc_c = acc_c + g * part
            out_ref[r0:r1, :] = acc_c

        for s in range(N_DEV - 1):
            c_send = (my - s) % N_DEV
            if s == 0:
                stage[0] = out_ref[pl.ds(c_send * CHUNK, CHUNK), :].astype(
                    jnp.bfloat16
                )
            rdma = pltpu.make_async_remote_copy(
                src_ref=stage.at[s],
                dst_ref=land.at[s],
                send_sem=rs_send.at[s],
                recv_sem=rs_recv.at[s],
                device_id=(right,),
                device_id_type=pl.DeviceIdType.MESH,
            )
            rdma.start()
            rdma.wait()
            c_recv = (my - 1 - s) % N_DEV
            merged = out_ref[pl.ds(c_recv * CHUNK, CHUNK), :] + land[s].astype(
                jnp.float32
            )
            out_ref[pl.ds(c_recv * CHUNK, CHUNK), :] = merged
            if s < N_DEV - 2:
                stage[s + 1] = merged.astype(jnp.bfloat16)

        q = (my + 1) % N_DEV
        shared_q = jnp.dot(
            x_ref[pl.ds(q * CHUNK, CHUNK), :],
            shared_ref[...],
            preferred_element_type=jnp.float32,
        )
        final_q = out_ref[pl.ds(q * CHUNK, CHUNK), :] + shared_q
        out_ref[pl.ds(q * CHUNK, CHUNK), :] = final_q
        ag_own[...] = final_q.astype(jnp.bfloat16)

        for s in range(N_DEV - 1):
            rdma = pltpu.make_async_remote_copy(
                src_ref=ag_own if s == 0 else ag_land.at[s - 1],
                dst_ref=ag_land.at[s],
                send_sem=ag_send.at[s],
                recv_sem=ag_recv.at[s],
                device_id=(right,),
                device_id_type=pl.DeviceIdType.MESH,
            )
            rdma.start()
            rdma.wait()
            c_recv = (my - s) % N_DEV
            out_ref[pl.ds(c_recv * CHUNK, CHUNK), :] = ag_land[s].astype(
                jnp.float32
            )

        @functools.partial(
            pl.run_scoped, second=pltpu.SemaphoreType.REGULAR
        )
        def _(second):
            for nbr in (left, right):
                pl.semaphore_signal(
                    second, inc=1, device_id=(nbr,),
                    device_id_type=pl.DeviceIdType.MESH,
                )
            pl.semaphore_wait(second, 2)

    return pl.pallas_call(
        body,
        out_shape=jax.ShapeDtypeStruct((N_TOK, D), jnp.float32),
        in_specs=[pl.BlockSpec(memory_space=pltpu.VMEM)] * 5,
        out_specs=pl.BlockSpec(memory_space=pltpu.VMEM),
        scratch_shapes=[
            pltpu.VMEM((N_DEV - 1, CHUNK, D), jnp.bfloat16),
            pltpu.VMEM((N_DEV - 1, CHUNK, D), jnp.bfloat16),
            pltpu.VMEM((CHUNK, D), jnp.bfloat16),
            pltpu.VMEM((N_DEV - 1, CHUNK, D), jnp.bfloat16),
            pltpu.SemaphoreType.DMA((N_DEV - 1,)),
            pltpu.SemaphoreType.DMA((N_DEV - 1,)),
            pltpu.SemaphoreType.DMA((N_DEV - 1,)),
            pltpu.SemaphoreType.DMA((N_DEV - 1,)),
        ],
        compiler_params=pltpu.CompilerParams(
            collective_id=0, vmem_limit_bytes=100 * 1024 * 1024
        ),
    )(x_bf, router_bf, route_idx, expert_bf, shared_bf)


# device time: 142269 ns/iter; 1.2025x vs baseline; 1.2025x over previous
import functools

import jax
import jax.numpy as jnp
from jax import lax
from jax.experimental import pallas as pl
from jax.experimental.pallas import tpu as pltpu

N_DEV = 4
N_TOK = 2048
D = 1024
E_LOCAL = 8
N_EXP = 32
CHUNK = N_TOK // N_DEV


def kernel(x, router_W, route_idx, expert_W, shared_W):
    x_bf = x.astype(jnp.bfloat16)
    router_bf = router_W.astype(jnp.bfloat16)
    expert_bf = expert_W.astype(jnp.bfloat16)
    shared_bf = shared_W.astype(jnp.bfloat16)

    def body(
        x_ref,
        router_ref,
        route_ref,
        expert_ref,
        shared_ref,
        out_ref,
        stage,
        land,
        ag_own,
        ag_land,
        rs_send,
        rs_recv,
        ag_send,
        ag_recv,
    ):
        my = lax.axis_index("i")
        left = (my - 1) % N_DEV
        right = (my + 1) % N_DEV

        barrier = pltpu.get_barrier_semaphore()
        for nbr in (left, right):
            pl.semaphore_signal(
                barrier, inc=1, device_id=(nbr,),
                device_id_type=pl.DeviceIdType.MESH,
            )
        pl.semaphore_wait(barrier, 2)

        col = lax.broadcasted_iota(jnp.int32, (CHUNK, N_EXP), 1)

        def expert_partial(c):
            rows = pl.ds(c * CHUNK, CHUNK)
            xc = x_ref[rows, :]
            scores = jnp.dot(
                xc, router_ref[...], preferred_element_type=jnp.float32
            )
            s_max = jnp.max(scores, axis=-1, keepdims=True)
            e_un = jnp.exp(scores - s_max)
            probs = e_un / jnp.sum(e_un, axis=-1, keepdims=True)
            routed = col == route_ref[rows, :]
            acc_c = jnp.zeros((CHUNK, D), jnp.float32)
            for le in range(E_LOCAL):
                e_glob = my * E_LOCAL + le
                g = jnp.sum(
                    jnp.where(routed & (col == e_glob), probs, 0.0),
                    axis=-1,
                    keepdims=True,
                )
                part = jnp.dot(
                    xc, expert_ref[le], preferred_element_type=jnp.float32
                )
                acc_c = acc_c + g * part
            return acc_c

        def rs_rdma(s):
            return pltpu.make_async_remote_copy(
                src_ref=stage.at[s],
                dst_ref=land.at[s],
                send_sem=rs_send.at[s],
                recv_sem=rs_recv.at[s],
                device_id=(right,),
                device_id_type=pl.DeviceIdType.MESH,
            )

        handles = []
        acc = expert_partial(my % N_DEV)
        stage[0] = acc.astype(jnp.bfloat16)
        r = rs_rdma(0)
        r.start()
        handles.append(r)
        for s in range(1, N_DEV):
            c = (my - s) % N_DEV
            acc = expert_partial(c)
            handles[s - 1].wait_recv()
            merged = acc + land[s - 1].astype(jnp.float32)
            if s < N_DEV - 1:
                stage[s] = merged.astype(jnp.bfloat16)
                r = rs_rdma(s)
                r.start()
                handles.append(r)
        q = (my + 1) % N_DEV
        q_rows = pl.ds(q * CHUNK, CHUNK)
        shared_q = jnp.dot(
            x_ref[q_rows, :], shared_ref[...],
            preferred_element_type=jnp.float32,
        )
        final_q = merged + shared_q
        out_ref[q_rows, :] = final_q
        ag_own[...] = final_q.astype(jnp.bfloat16)

        ag_handles = []
        for s in range(N_DEV - 1):
            if s > 0:
                ag_handles[s - 1].wait_recv()
            r = pltpu.make_async_remote_copy(
                src_ref=ag_own if s == 0 else ag_land.at[s - 1],
                dst_ref=ag_land.at[s],
                send_sem=ag_send.at[s],
                recv_sem=ag_recv.at[s],
                device_id=(right,),
                device_id_type=pl.DeviceIdType.MESH,
            )
            r.start()
            ag_handles.append(r)
            if s > 0:
                c_prev = (my - (s - 1)) % N_DEV
                out_ref[pl.ds(c_prev * CHUNK, CHUNK), :] = ag_land[
                    s - 1
                ].astype(jnp.float32)
        ag_handles[N_DEV - 2].wait_recv()
        c_last = (my - (N_DEV - 2)) % N_DEV
        out_ref[pl.ds(c_last * CHUNK, CHUNK), :] = ag_land[N_DEV - 2].astype(
            jnp.float32
        )
        handles.extend(ag_handles)

        for r in handles:
            r.wait_send()

        @functools.partial(pl.run_scoped, second=pltpu.SemaphoreType.REGULAR)
        def _(second):
            for nbr in (left, right):
                pl.semaphore_signal(
                    second, inc=1, device_id=(nbr,),
                    device_id_type=pl.DeviceIdType.MESH,
                )
            pl.semaphore_wait(second, 2)

    return pl.pallas_call(
        body,
        out_shape=jax.ShapeDtypeStruct((N_TOK, D), jnp.float32),
        in_specs=[pl.BlockSpec(memory_space=pltpu.VMEM)] * 5,
        out_specs=pl.BlockSpec(memory_space=pltpu.VMEM),
        scratch_shapes=[
            pltpu.VMEM((N_DEV - 1, CHUNK, D), jnp.bfloat16),
            pltpu.VMEM((N_DEV - 1, CHUNK, D), jnp.bfloat16),
            pltpu.VMEM((CHUNK, D), jnp.bfloat16),
            pltpu.VMEM((N_DEV - 1, CHUNK, D), jnp.bfloat16),
            pltpu.SemaphoreType.DMA((N_DEV - 1,)),
            pltpu.SemaphoreType.DMA((N_DEV - 1,)),
            pltpu.SemaphoreType.DMA((N_DEV - 1,)),
            pltpu.SemaphoreType.DMA((N_DEV - 1,)),
        ],
        compiler_params=pltpu.CompilerParams(
            collective_id=0, vmem_limit_bytes=100 * 1024 * 1024
        ),
    )(x_bf, router_bf, route_idx, expert_bf, shared_bf)


# device time: 124363 ns/iter; 1.3756x vs baseline; 1.1440x over previous
import functools

import jax
import jax.numpy as jnp
from jax import lax
from jax.experimental import pallas as pl
from jax.experimental.pallas import tpu as pltpu

N_DEV = 4
N_TOK = 2048
D = 1024
E_LOCAL = 8
N_EXP = 32
CHUNK = N_TOK // N_DEV


def kernel(x, router_W, route_idx, expert_W, shared_W):
    x_bf = x.astype(jnp.bfloat16)
    router_bf = router_W.astype(jnp.bfloat16)
    expert_bf = expert_W.astype(jnp.bfloat16)
    shared_bf = shared_W.astype(jnp.bfloat16)

    def body(
        x_ref,
        router_ref,
        route_ref,
        expert_ref,
        shared_ref,
        out_ref,
        stage,
        land,
        ag_own_cw,
        ag_own_ccw,
        ag_land_cw,
        ag_land_ccw,
        rs_send,
        rs_recv,
        ag_send_cw,
        ag_recv_cw,
        ag_send_ccw,
        ag_recv_ccw,
    ):
        my = lax.axis_index("i")
        left = (my - 1) % N_DEV
        right = (my + 1) % N_DEV

        barrier = pltpu.get_barrier_semaphore()
        for nbr in (left, right):
            pl.semaphore_signal(
                barrier, inc=1, device_id=(nbr,),
                device_id_type=pl.DeviceIdType.MESH,
            )
        pl.semaphore_wait(barrier, 2)

        col = lax.broadcasted_iota(jnp.int32, (CHUNK, N_EXP), 1)

        def expert_partial(c):
            rows = pl.ds(c * CHUNK, CHUNK)
            xc = x_ref[rows, :]
            scores = jnp.dot(
                xc, router_ref[...], preferred_element_type=jnp.float32
            )
            s_max = jnp.max(scores, axis=-1, keepdims=True)
            e_un = jnp.exp(scores - s_max)
            probs = e_un / jnp.sum(e_un, axis=-1, keepdims=True)
            routed = col == route_ref[rows, :]
            acc_c = jnp.zeros((CHUNK, D), jnp.float32)
            for le in range(E_LOCAL):
                e_glob = my * E_LOCAL + le
                g = jnp.sum(
                    jnp.where(routed & (col == e_glob), probs, 0.0),
                    axis=-1,
                    keepdims=True,
                )
                part = jnp.dot(
                    xc, expert_ref[le], preferred_element_type=jnp.float32
                )
                acc_c = acc_c + g * part
            return acc_c

        def rs_rdma(s):
            return pltpu.make_async_remote_copy(
                src_ref=stage.at[s],
                dst_ref=land.at[s],
                send_sem=rs_send.at[s],
                recv_sem=rs_recv.at[s],
                device_id=(right,),
                device_id_type=pl.DeviceIdType.MESH,
            )

        handles = []
        acc = expert_partial(my % N_DEV)
        stage[0] = acc.astype(jnp.bfloat16)
        r = rs_rdma(0)
        r.start()
        handles.append(r)
        q = (my + 1) % N_DEV
        q_rows = pl.ds(q * CHUNK, CHUNK)
        shared_q = None
        for s in range(1, N_DEV):
            c = (my - s) % N_DEV
            acc = expert_partial(c)
            if s == N_DEV - 1:
                shared_q = jnp.dot(
                    x_ref[q_rows, :], shared_ref[...],
                    preferred_element_type=jnp.float32,
                )
            handles[s - 1].wait_recv()
            merged = acc + land[s - 1].astype(jnp.float32)
            if s < N_DEV - 1:
                stage[s] = merged.astype(jnp.bfloat16)
                r = rs_rdma(s)
                r.start()
                handles.append(r)
        final_q = merged + shared_q
        out_ref[q_rows, :] = final_q
        fin_bf = final_q.astype(jnp.bfloat16)
        ag_own_cw[...] = fin_bf[:, 0 : D // 2]
        ag_own_ccw[...] = fin_bf[:, D // 2 : D]

        H = D // 2
        ag_cw, ag_ccw = [], []
        for s in range(N_DEV - 1):
            if s > 0:
                ag_cw[s - 1].wait_recv()
                ag_ccw[s - 1].wait_recv()
            r_cw = pltpu.make_async_remote_copy(
                src_ref=ag_own_cw if s == 0 else ag_land_cw.at[s - 1],
                dst_ref=ag_land_cw.at[s],
                send_sem=ag_send_cw.at[s],
                recv_sem=ag_recv_cw.at[s],
                device_id=(right,),
                device_id_type=pl.DeviceIdType.MESH,
            )
            r_cw.start()
            ag_cw.append(r_cw)
            r_ccw = pltpu.make_async_remote_copy(
                src_ref=ag_own_ccw if s == 0 else ag_land_ccw.at[s - 1],
                dst_ref=ag_land_ccw.at[s],
                send_sem=ag_send_ccw.at[s],
                recv_sem=ag_recv_ccw.at[s],
                device_id=(left,),
                device_id_type=pl.DeviceIdType.MESH,
            )
            r_ccw.start()
            ag_ccw.append(r_ccw)
            if s > 0:
                c_cw = (my - (s - 1)) % N_DEV
                out_ref[pl.ds(c_cw * CHUNK, CHUNK), 0:H] = ag_land_cw[
                    s - 1
                ].astype(jnp.float32)
                c_ccw = (my + 2 + (s - 1)) % N_DEV
                out_ref[pl.ds(c_ccw * CHUNK, CHUNK), H:D] = ag_land_ccw[
                    s - 1
                ].astype(jnp.float32)
        last = N_DEV - 2
        ag_cw[last].wait_recv()
        ag_ccw[last].wait_recv()
        c_cw = (my - last) % N_DEV
        out_ref[pl.ds(c_cw * CHUNK, CHUNK), 0:H] = ag_land_cw[last].astype(
            jnp.float32
        )
        c_ccw = (my + 2 + last) % N_DEV
        out_ref[pl.ds(c_ccw * CHUNK, CHUNK), H:D] = ag_land_ccw[last].astype(
            jnp.float32
        )
        handles.extend(ag_cw)
        handles.extend(ag_ccw)

        for r in handles:
            r.wait_send()

        @functools.partial(pl.run_scoped, second=pltpu.SemaphoreType.REGULAR)
        def _(second):
            for nbr in (left, right):
                pl.semaphore_signal(
                    second, inc=1, device_id=(nbr,),
                    device_id_type=pl.DeviceIdType.MESH,
                )
            pl.semaphore_wait(second, 2)

    return pl.pallas_call(
        body,
        out_shape=jax.ShapeDtypeStruct((N_TOK, D), jnp.float32),
        in_specs=[pl.BlockSpec(memory_space=pltpu.VMEM)] * 5,
        out_specs=pl.BlockSpec(memory_space=pltpu.VMEM),
        scratch_shapes=[
            pltpu.VMEM((N_DEV - 1, CHUNK, D), jnp.bfloat16),
            pltpu.VMEM((N_DEV - 1, CHUNK, D), jnp.bfloat16),
            pltpu.VMEM((CHUNK, D // 2), jnp.bfloat16),
            pltpu.VMEM((CHUNK, D // 2), jnp.bfloat16),
            pltpu.VMEM((N_DEV - 1, CHUNK, D // 2), jnp.bfloat16),
            pltpu.VMEM((N_DEV - 1, CHUNK, D // 2), jnp.bfloat16),
            pltpu.SemaphoreType.DMA((N_DEV - 1,)),
            pltpu.SemaphoreType.DMA((N_DEV - 1,)),
            pltpu.SemaphoreType.DMA((N_DEV - 1,)),
            pltpu.SemaphoreType.DMA((N_DEV - 1,)),
            pltpu.SemaphoreType.DMA((N_DEV - 1,)),
            pltpu.SemaphoreType.DMA((N_DEV - 1,)),
        ],
        compiler_params=pltpu.CompilerParams(
            collective_id=0, vmem_limit_bytes=100 * 1024 * 1024
        ),
    )(x_bf, router_bf, route_idx, expert_bf, shared_bf)
